# baseline (speedup 1.0000x reference)
import functools

import jax
import jax.numpy as jnp
from jax import lax
from jax.experimental import pallas as pl
from jax.experimental.pallas import tpu as pltpu
from jax.experimental.pallas import tpu_sc as plsc

NSEG = 512
N_NODES = 100000
D_FEAT = 128
LANES = 16

NUM_CORES = 2
NUM_SUBCORES = 16
NUM_WORKERS = NUM_CORES * NUM_SUBCORES

CHUNK = 80
NUM_CHUNKS = N_NODES // CHUNK
ITERS = (NUM_CHUNKS + NUM_WORKERS - 1) // NUM_WORKERS
ROWS_PER_TILE = NSEG // NUM_SUBCORES


def _sc_partial_sums(x, batch):
    mesh = plsc.VectorSubcoreMesh(core_axis_name="c", subcore_axis_name="s")

    @functools.partial(
        pl.kernel,
        mesh=mesh,
        out_type=(
            jax.ShapeDtypeStruct((NUM_CORES, NSEG, D_FEAT), jnp.float32),
            jax.ShapeDtypeStruct((NUM_CORES, NSEG, D_FEAT), jnp.float32),
        ),
        scratch_types=[
            pltpu.VMEM_SHARED((NSEG, D_FEAT), jnp.float32),
            pltpu.VMEM_SHARED((NSEG, D_FEAT), jnp.float32),
            pltpu.VMEM((CHUNK, D_FEAT), jnp.float32),
            pltpu.VMEM((CHUNK, D_FEAT), jnp.float32),
            pltpu.VMEM((CHUNK, D_FEAT), jnp.float32),
            pltpu.VMEM((CHUNK, D_FEAT), jnp.float32),
            pltpu.VMEM((CHUNK, D_FEAT), jnp.float32),
            pltpu.VMEM((CHUNK, D_FEAT), jnp.float32),
            pltpu.VMEM((CHUNK, D_FEAT), jnp.float32),
            pltpu.VMEM((CHUNK, D_FEAT), jnp.float32),
            pltpu.VMEM((CHUNK,), jnp.int32),
            pltpu.VMEM((CHUNK,), jnp.int32),
            pltpu.VMEM((CHUNK,), jnp.int32),
            pltpu.VMEM((CHUNK,), jnp.int32),
            pltpu.VMEM((CHUNK,), jnp.int32),
            pltpu.VMEM((CHUNK,), jnp.int32),
            pltpu.VMEM((CHUNK,), jnp.int32),
            pltpu.VMEM((CHUNK,), jnp.int32),
            pltpu.VMEM((CHUNK, D_FEAT), jnp.float32),
            pltpu.VMEM((ROWS_PER_TILE, D_FEAT), jnp.float32),
            pltpu.SemaphoreType.DMA,
            pltpu.SemaphoreType.DMA,
            pltpu.SemaphoreType.DMA,
            pltpu.SemaphoreType.DMA,
            pltpu.SemaphoreType.DMA,
            pltpu.SemaphoreType.DMA,
            pltpu.SemaphoreType.DMA,
            pltpu.SemaphoreType.DMA,
            pltpu.SemaphoreType.DMA,
            pltpu.SemaphoreType.DMA,
            pltpu.SemaphoreType.DMA,
            pltpu.SemaphoreType.DMA,
            pltpu.SemaphoreType.DMA,
            pltpu.SemaphoreType.DMA,
            pltpu.SemaphoreType.DMA,
            pltpu.SemaphoreType.DMA,
        ],
    )
    def sc_kernel(x_hbm, b_hbm, sums_hbm, cnt_hbm,
                  acc_sh, cnt_sh, xb0, xb1, xb2, xb3, xb4, xb5, xb6, xb7,
                  ib0, ib1, ib2, ib3, ib4, ib5, ib6, ib7, onesb, outb,
                  ld0, ld1, ld2, ld3, ld4, ld5, ld6, ld7,
                  sc0, sc1, sc2, sc3, sc4, sc5, sc6, sc7):
        c = lax.axis_index("c")
        s = lax.axis_index("s")
        w = s * NUM_CORES + c

        zeros16 = jnp.zeros((LANES,), jnp.float32)
        ones16 = jnp.ones((LANES,), jnp.float32)

        slots = ((xb0, ib0, ld0, sc0), (xb1, ib1, ld1, sc1),
                 (xb2, ib2, ld2, sc2), (xb3, ib3, ld3, sc3),
                 (xb4, ib4, ld4, sc4), (xb5, ib5, ld5, sc5),
                 (xb6, ib6, ld6, sc6), (xb7, ib7, ld7, sc7))

        def start_load(i, xbuf, ibuf, sem):
            chunk = i * NUM_WORKERS + w

            @pl.when(chunk < NUM_CHUNKS)
            def _():
                base = chunk * CHUNK
                pltpu.async_copy(x_hbm.at[pl.ds(base, CHUNK)], xbuf, sem)
                pltpu.async_copy(b_hbm.at[pl.ds(base, CHUNK)], ibuf, sem)

        def wait_load(i, xbuf, ibuf, sem):
            chunk = i * NUM_WORKERS + w

            @pl.when(chunk < NUM_CHUNKS)
            def _():
                pltpu.make_async_copy(x_hbm.at[pl.ds(0, CHUNK)], xbuf, sem).wait()
                pltpu.make_async_copy(b_hbm.at[pl.ds(0, CHUNK)], ibuf, sem).wait()

        def drain_scatter(i, xbuf, ibuf, sem):
            chunk = i * NUM_WORKERS + w

            @pl.when((i >= 0) & (chunk < NUM_CHUNKS))
            def _():
                pltpu.make_async_copy(xbuf, acc_sh.at[ibuf], sem).wait()
                pltpu.make_async_copy(onesb, cnt_sh.at[ibuf], sem).wait()

        for j in range(ROWS_PER_TILE):
            for t in range(D_FEAT // LANES):
                outb[j, pl.ds(t * LANES, LANES)] = zeros16
        pltpu.sync_copy(outb, acc_sh.at[pl.ds(s * ROWS_PER_TILE, ROWS_PER_TILE)])
        pltpu.sync_copy(outb, cnt_sh.at[pl.ds(s * ROWS_PER_TILE, ROWS_PER_TILE)])

        for j in range(CHUNK):
            for t in range(D_FEAT // LANES):
                onesb[j, pl.ds(t * LANES, LANES)] = ones16

        plsc.subcore_barrier()

        start_load(0, xb0, ib0, ld0)
        start_load(1, xb1, ib1, ld1)
        start_load(2, xb2, ib2, ld2)
        start_load(3, xb3, ib3, ld3)

        def body(j, carry):
            for b, (xbuf, ibuf, lds, scs) in enumerate(slots):
                i = j * 8 + b
                chunk = i * NUM_WORKERS + w
                wait_load(i, xbuf, ibuf, lds)

                @pl.when(chunk < NUM_CHUNKS)
                def _():
                    pltpu.async_copy(xbuf, acc_sh.at[ibuf], scs, add=True)
                    pltpu.async_copy(onesb, cnt_sh.at[ibuf], scs, add=True)

                pb = (b + 4) % 8
                xprev, iprev, ldprev, scprev = slots[pb]
                drain_scatter(i - 4, xprev, iprev, scprev)
                start_load(i + 4, xprev, iprev, ldprev)
            return carry

        lax.fori_loop(0, ITERS // 8, body, 0)

        for k in (ITERS - 4, ITERS - 3, ITERS - 2, ITERS - 1):
            xbuf, ibuf, _, scs = slots[k % 8]
            drain_scatter(k, xbuf, ibuf, scs)

        plsc.subcore_barrier()

        row0 = s * ROWS_PER_TILE
        pltpu.sync_copy(acc_sh.at[pl.ds(row0, ROWS_PER_TILE)], outb)
        pltpu.sync_copy(outb, sums_hbm.at[c, pl.ds(row0, ROWS_PER_TILE)])
        pltpu.sync_copy(cnt_sh.at[pl.ds(row0, ROWS_PER_TILE)], outb)
        pltpu.sync_copy(outb, cnt_hbm.at[c, pl.ds(row0, ROWS_PER_TILE)])

    return sc_kernel(x, batch)


def _merge(sums, cnts):
    def body(s_ref, c_ref, o_ref):
        total = s_ref[0] + s_ref[1]
        cnt = c_ref[0, :, 0:1] + c_ref[1, :, 0:1]
        o_ref[...] = total / jnp.maximum(cnt, 1.0)

    return pl.pallas_call(
        body,
        out_shape=jax.ShapeDtypeStruct((NSEG, D_FEAT), jnp.float32),
    )(sums, cnts)


def kernel(x, edge_index, batch):
    del edge_index
    sums, cnts = _sc_partial_sums(x, batch.astype(jnp.int32))
    return _merge(sums, cnts)


# speedup vs baseline: 1.0439x; 1.0439x over previous
import functools

import jax
import jax.numpy as jnp
from jax import lax
from jax.experimental import pallas as pl
from jax.experimental.pallas import tpu as pltpu
from jax.experimental.pallas import tpu_sc as plsc

NSEG = 512
N_NODES = 100000
D_FEAT = 128
LANES = 16

NUM_CORES = 2
NUM_SUBCORES = 16
NUM_WORKERS = NUM_CORES * NUM_SUBCORES

CHUNK = 80
NUM_CHUNKS = N_NODES // CHUNK
ITERS = (NUM_CHUNKS + NUM_WORKERS - 1) // NUM_WORKERS
ROWS_PER_TILE = NSEG // NUM_SUBCORES


def _sc_partial_sums(x, batch):
    mesh = plsc.VectorSubcoreMesh(core_axis_name="c", subcore_axis_name="s")

    @functools.partial(
        pl.kernel,
        mesh=mesh,
        out_type=(
            jax.ShapeDtypeStruct((NUM_CORES, NSEG, D_FEAT), jnp.float32),
            jax.ShapeDtypeStruct((NUM_CORES, NSEG, D_FEAT), jnp.float32),
        ),
        scratch_types=[
            pltpu.VMEM_SHARED((NSEG, D_FEAT), jnp.float32),
            pltpu.VMEM_SHARED((NSEG, D_FEAT), jnp.float32),
            pltpu.VMEM((CHUNK, D_FEAT), jnp.float32),
            pltpu.VMEM((CHUNK, D_FEAT), jnp.float32),
            pltpu.VMEM((CHUNK, D_FEAT), jnp.float32),
            pltpu.VMEM((CHUNK, D_FEAT), jnp.float32),
            pltpu.VMEM((CHUNK,), jnp.int32),
            pltpu.VMEM((CHUNK,), jnp.int32),
            pltpu.VMEM((CHUNK,), jnp.int32),
            pltpu.VMEM((CHUNK,), jnp.int32),
            pltpu.VMEM((CHUNK, D_FEAT), jnp.float32),
            pltpu.VMEM((ROWS_PER_TILE, D_FEAT), jnp.float32),
            pltpu.SemaphoreType.DMA,
            pltpu.SemaphoreType.DMA,
            pltpu.SemaphoreType.DMA,
            pltpu.SemaphoreType.DMA,
            pltpu.SemaphoreType.DMA,
            pltpu.SemaphoreType.DMA,
            pltpu.SemaphoreType.DMA,
            pltpu.SemaphoreType.DMA,
        ],
    )
    def sc_kernel(x_hbm, b_hbm, sums_hbm, cnt_hbm,
                  acc_sh, cnt_sh, xb0, xb1, xb2, xb3, ib0, ib1, ib2, ib3,
                  onesb, outb, ld0, ld1, ld2, ld3, sc0, sc1, sc2, sc3):
        c = lax.axis_index("c")
        s = lax.axis_index("s")
        w = s * NUM_CORES + c

        zeros16 = jnp.zeros((LANES,), jnp.float32)
        ones16 = jnp.ones((LANES,), jnp.float32)

        slots = ((xb0, ib0, ld0, sc0), (xb1, ib1, ld1, sc1),
                 (xb2, ib2, ld2, sc2), (xb3, ib3, ld3, sc3))

        def start_load(i, xbuf, ibuf, sem):
            chunk = i * NUM_WORKERS + w

            @pl.when(chunk < NUM_CHUNKS)
            def _():
                base = chunk * CHUNK
                pltpu.async_copy(x_hbm.at[pl.ds(base, CHUNK)], xbuf, sem)
                pltpu.async_copy(b_hbm.at[pl.ds(base, CHUNK)], ibuf, sem)

        def wait_load(i, xbuf, ibuf, sem):
            chunk = i * NUM_WORKERS + w

            @pl.when(chunk < NUM_CHUNKS)
            def _():
                pltpu.make_async_copy(x_hbm.at[pl.ds(0, CHUNK)], xbuf, sem).wait()
                pltpu.make_async_copy(b_hbm.at[pl.ds(0, CHUNK)], ibuf, sem).wait()

        def drain_scatter(i, xbuf, ibuf, sem):
            chunk = i * NUM_WORKERS + w

            @pl.when((i >= 0) & (chunk < NUM_CHUNKS))
            def _():
                pltpu.make_async_copy(xbuf, acc_sh.at[ibuf], sem).wait()
                pltpu.make_async_copy(onesb, cnt_sh.at[ibuf], sem).wait()

        for j in range(ROWS_PER_TILE):
            for t in range(D_FEAT // LANES):
                outb[j, pl.ds(t * LANES, LANES)] = zeros16
        pltpu.sync_copy(outb, acc_sh.at[pl.ds(s * ROWS_PER_TILE, ROWS_PER_TILE)])
        pltpu.sync_copy(outb, cnt_sh.at[pl.ds(s * ROWS_PER_TILE, ROWS_PER_TILE)])

        for j in range(CHUNK):
            for t in range(D_FEAT // LANES):
                onesb[j, pl.ds(t * LANES, LANES)] = ones16

        plsc.subcore_barrier()

        start_load(0, xb0, ib0, ld0)
        start_load(1, xb1, ib1, ld1)

        def body(j, carry):
            for b, (xbuf, ibuf, lds, scs) in enumerate(slots):
                i = j * 4 + b
                chunk = i * NUM_WORKERS + w
                wait_load(i, xbuf, ibuf, lds)

                @pl.when(chunk < NUM_CHUNKS)
                def _():
                    pltpu.async_copy(xbuf, acc_sh.at[ibuf], scs, add=True)
                    pltpu.async_copy(onesb, cnt_sh.at[ibuf], scs, add=True)

                pb = (b + 2) % 4
                xprev, iprev, ldprev, scprev = slots[pb]
                drain_scatter(i - 2, xprev, iprev, scprev)
                start_load(i + 2, xprev, iprev, ldprev)
            return carry

        lax.fori_loop(0, ITERS // 4, body, 0)

        for k in (ITERS - 2, ITERS - 1):
            xbuf, ibuf, _, scs = slots[k % 4]
            drain_scatter(k, xbuf, ibuf, scs)

        plsc.subcore_barrier()

        row0 = s * ROWS_PER_TILE
        pltpu.sync_copy(acc_sh.at[pl.ds(row0, ROWS_PER_TILE)], outb)
        pltpu.sync_copy(outb, sums_hbm.at[c, pl.ds(row0, ROWS_PER_TILE)])
        pltpu.sync_copy(cnt_sh.at[pl.ds(row0, ROWS_PER_TILE)], outb)
        pltpu.sync_copy(outb, cnt_hbm.at[c, pl.ds(row0, ROWS_PER_TILE)])

    return sc_kernel(x, batch)


def _merge(sums, cnts):
    def body(s_ref, c_ref, o_ref):
        total = s_ref[0] + s_ref[1]
        cnt = c_ref[0, :, 0:1] + c_ref[1, :, 0:1]
        o_ref[...] = total / jnp.maximum(cnt, 1.0)

    return pl.pallas_call(
        body,
        out_shape=jax.ShapeDtypeStruct((NSEG, D_FEAT), jnp.float32),
    )(sums, cnts)


def kernel(x, edge_index, batch):
    del edge_index
    sums, cnts = _sc_partial_sums(x, batch.astype(jnp.int32))
    return _merge(sums, cnts)
